# Initial kernel scaffold; baseline (speedup 1.0000x reference)
import jax
import jax.numpy as jnp
from jax.experimental import pallas as pl


def kernel(e_prev, edge_index, W1, b1, gamma1, beta1, W2, b2, gamma2, beta2):
    raise NotImplementedError("write your pallas kernel here")


# speedup vs baseline: 8.1012x; 8.1012x over previous
import functools

import jax
import jax.numpy as jnp
from jax import lax
from jax.experimental import pallas as pl
from jax.experimental.pallas import tpu as pltpu
from jax.experimental.pallas import tpu_sc as plsc

N = 10000
E = 320000
D = 128
NC = 2
NS = 16
NW = NC * NS
CHUNK = 128
CPT = 80
EPAD = NW * CPT * CHUNK
RPT = 640
NPAD = NS * RPT
DUMMY = N
DEGW = 128

_mesh = plsc.VectorSubcoreMesh(core_axis_name="c", subcore_axis_name="s")


@functools.partial(
    pl.kernel,
    out_type=jax.ShapeDtypeStruct((NC, NPAD, DEGW), jnp.float32),
    mesh=_mesh,
    scratch_types=[
        pltpu.VMEM((CPT, CHUNK), jnp.int32),
        pltpu.VMEM((CHUNK, DEGW), jnp.float32),
        pltpu.VMEM_SHARED((NPAD, DEGW), jnp.float32),
    ],
)
def _deg_kernel(dst_hbm, ones_hbm, zeros_hbm, out_hbm, dst_v, ones_v, acc):
    cid = lax.axis_index("c")
    sid = lax.axis_index("s")
    wid = cid * NS + sid
    pltpu.sync_copy(zeros_hbm, acc.at[pl.ds(sid * RPT, RPT)])
    pltpu.sync_copy(ones_hbm, ones_v)
    pltpu.sync_copy(dst_hbm.at[wid], dst_v)
    plsc.subcore_barrier()

    def body(j, carry):
        pltpu.sync_copy(ones_v, acc.at[dst_v.at[j]], add=True)
        return carry

    lax.fori_loop(0, CPT, body, 0)
    plsc.subcore_barrier()
    pltpu.sync_copy(acc.at[pl.ds(sid * RPT, RPT)],
                    out_hbm.at[cid, pl.ds(sid * RPT, RPT)])


@functools.partial(
    pl.kernel,
    out_type=jax.ShapeDtypeStruct((NC, NPAD, D), jnp.float32),
    mesh=_mesh,
    scratch_types=[
        pltpu.VMEM((CPT, CHUNK), jnp.int32),
        pltpu.VMEM((CPT, CHUNK), jnp.int32),
        pltpu.VMEM((CHUNK, D), jnp.float32),
        pltpu.VMEM_SHARED((NPAD, D), jnp.float32),
        pltpu.SemaphoreType.DMA,
    ],
)
def _agg_kernel(hs_hbm, src_hbm, dst_hbm, zrows_hbm, out_hbm,
                src_v, dst_v, rows_v, acc, sem):
    cid = lax.axis_index("c")
    sid = lax.axis_index("s")
    wid = cid * NS + sid
    pltpu.sync_copy(zrows_hbm, acc.at[pl.ds(sid * RPT, RPT)])
    pltpu.sync_copy(src_hbm.at[wid], src_v)
    pltpu.sync_copy(dst_hbm.at[wid], dst_v)
    plsc.subcore_barrier()

    def body(j, carry):
        pltpu.async_copy(hs_hbm.at[src_v.at[j]], rows_v, sem).wait()
        pltpu.sync_copy(rows_v, acc.at[dst_v.at[j]], add=True)
        return carry

    lax.fori_loop(0, CPT, body, 0)
    plsc.subcore_barrier()
    pltpu.sync_copy(acc.at[pl.ds(sid * RPT, RPT)],
                    out_hbm.at[cid, pl.ds(sid * RPT, RPT)])


def _tc1_body(degp_ref, x_ref, w1_ref, dinv_ref, hs_ref):
    deg = degp_ref[0, :N, 0:1] + degp_ref[1, :N, 0:1] + 1.0
    dinv = lax.rsqrt(deg)
    dinv_ref[...] = dinv
    h = jnp.dot(x_ref[...], w1_ref[...], preferred_element_type=jnp.float32)
    hs_ref[...] = h * dinv


def _tc2_body(p_ref, hs_ref, dinv_ref, b_ref, g_ref, be_ref, w2_ref, out_ref):
    dinv = dinv_ref[...]
    acc = p_ref[0, :N, :] + p_ref[1, :N, :] + hs_ref[...]
    y = acc * dinv + b_ref[...]
    mean = jnp.mean(y, axis=0, keepdims=True)
    var = jnp.mean((y - mean) * (y - mean), axis=0, keepdims=True)
    yn = (y - mean) * lax.rsqrt(var + 1e-5) * g_ref[...] + be_ref[...]
    yr = jnp.maximum(yn, 0.0)
    h2 = jnp.dot(yr, w2_ref[...], preferred_element_type=jnp.float32)
    out_ref[...] = h2 * dinv


def _tc3_body(p_ref, hs_ref, dinv_ref, b_ref, g_ref, be_ref, out_ref):
    acc = p_ref[0, :N, :] + p_ref[1, :N, :] + hs_ref[...]
    y = acc * dinv_ref[...] + b_ref[...]
    mean = jnp.mean(y, axis=0, keepdims=True)
    var = jnp.mean((y - mean) * (y - mean), axis=0, keepdims=True)
    out_ref[...] = (y - mean) * lax.rsqrt(var + 1e-5) * g_ref[...] + be_ref[...]


_f32 = jnp.float32
_tc1 = pl.pallas_call(
    _tc1_body,
    out_shape=[jax.ShapeDtypeStruct((N, 1), _f32),
               jax.ShapeDtypeStruct((N, D), _f32)],
)
_tc2 = pl.pallas_call(
    _tc2_body,
    out_shape=jax.ShapeDtypeStruct((N, D), _f32),
)
_tc3 = pl.pallas_call(
    _tc3_body,
    out_shape=jax.ShapeDtypeStruct((N, D), _f32),
)


def kernel(e_prev, edge_index, W1, b1, gamma1, beta1, W2, b2, gamma2, beta2):
    src = edge_index[0]
    dst = edge_index[1]
    pad = EPAD - E
    srcp = jnp.concatenate([src, jnp.zeros((pad,), jnp.int32)])
    srcp = srcp.reshape(NW, CPT, CHUNK)
    dstp = jnp.concatenate([dst, jnp.full((pad,), DUMMY, jnp.int32)])
    dstp = dstp.reshape(NW, CPT, CHUNK)

    ones_c = jnp.ones((CHUNK, DEGW), _f32)
    zeros_r = jnp.zeros((RPT, DEGW), _f32)
    zrows = jnp.zeros((RPT, D), _f32)

    degp = _deg_kernel(dstp, ones_c, zeros_r)
    dinv, hs1 = _tc1(degp, e_prev, W1)
    p1 = _agg_kernel(hs1, srcp, dstp, zrows)
    hs2 = _tc2(p1, hs1, dinv, b1.reshape(1, D), gamma1.reshape(1, D),
               beta1.reshape(1, D), W2)
    p2 = _agg_kernel(hs2, srcp, dstp, zrows)
    out = _tc3(p2, hs2, dinv, b2.reshape(1, D), gamma2.reshape(1, D),
               beta2.reshape(1, D))
    return out


# speedup vs baseline: 9.0205x; 1.1135x over previous
import functools

import jax
import jax.numpy as jnp
from jax import lax
from jax.experimental import pallas as pl
from jax.experimental.pallas import tpu as pltpu
from jax.experimental.pallas import tpu_sc as plsc

N = 10000
E = 320000
D = 128
NC = 2
NS = 16
NW = NC * NS
CHUNK = 128
CPT = 80
EPAD = NW * CPT * CHUNK
RPT = 640
NPAD = NS * RPT
DUMMY = N
DEGW = 128

_mesh = plsc.VectorSubcoreMesh(core_axis_name="c", subcore_axis_name="s")


@functools.partial(
    pl.kernel,
    out_type=jax.ShapeDtypeStruct((NC, NPAD, DEGW), jnp.float32),
    mesh=_mesh,
    scratch_types=[
        pltpu.VMEM((CPT, CHUNK), jnp.int32),
        pltpu.VMEM((CHUNK, DEGW), jnp.float32),
        pltpu.VMEM_SHARED((NPAD, DEGW), jnp.float32),
        pltpu.SemaphoreType.DMA,
    ],
)
def _deg_kernel(dst_hbm, ones_hbm, zeros_hbm, out_hbm, dst_v, ones_v, acc, sem):
    cid = lax.axis_index("c")
    sid = lax.axis_index("s")
    wid = cid * NS + sid
    pltpu.sync_copy(zeros_hbm, acc.at[pl.ds(sid * RPT, RPT)])
    pltpu.sync_copy(ones_hbm, ones_v)
    pltpu.sync_copy(dst_hbm.at[wid], dst_v)
    plsc.subcore_barrier()

    def fire(j, carry):
        pltpu.async_copy(ones_v, acc.at[dst_v.at[j]], sem, add=True)
        return carry

    lax.fori_loop(0, CPT, fire, 0)

    def drain(j, carry):
        pltpu.make_async_copy(ones_v, acc.at[dst_v.at[j]], sem).wait()
        return carry

    lax.fori_loop(0, CPT, drain, 0)
    plsc.subcore_barrier()
    pltpu.sync_copy(acc.at[pl.ds(sid * RPT, RPT)],
                    out_hbm.at[cid, pl.ds(sid * RPT, RPT)])


NBUF = 2
NHALF = 2
HCPT = CPT // NHALF


@functools.partial(
    pl.kernel,
    out_type=jax.ShapeDtypeStruct((NC, NPAD, D), jnp.float32),
    mesh=_mesh,
    scratch_types=(
        [pltpu.VMEM_SHARED((NPAD, D), jnp.float32)]
        + [pltpu.VMEM((HCPT, CHUNK), jnp.int32)] * 2
        + [pltpu.VMEM((CHUNK, D), jnp.float32)] * NBUF
        + [pltpu.SemaphoreType.DMA] * (2 * NBUF)
    ),
)
def _agg_kernel(hs_hbm, src_hbm, dst_hbm, zrows_hbm, out_hbm,
                acc, src_v, dst_v, r0, r1, g0, g1, s0, s1):
    rows = (r0, r1)
    gsem = (g0, g1)
    ssem = (s0, s1)
    cid = lax.axis_index("c")
    sid = lax.axis_index("s")
    wid = cid * NS + sid

    def fire_gather(j, b):
        pltpu.async_copy(hs_hbm.at[src_v.at[j]], rows[b], gsem[b])

    def wait_gather(j, b):
        pltpu.make_async_copy(hs_hbm.at[src_v.at[j]], rows[b], gsem[b]).wait()

    def fire_scatter(j, b):
        pltpu.async_copy(rows[b], acc.at[dst_v.at[j]], ssem[b], add=True)

    def wait_scatter(j, b):
        pltpu.make_async_copy(rows[b], acc.at[dst_v.at[j]], ssem[b]).wait()

    pltpu.sync_copy(zrows_hbm, acc.at[pl.ds(sid * RPT, RPT)])
    plsc.subcore_barrier()

    for h in range(NHALF):
        pltpu.sync_copy(src_hbm.at[wid, pl.ds(h * HCPT, HCPT)], src_v)
        pltpu.sync_copy(dst_hbm.at[wid, pl.ds(h * HCPT, HCPT)], dst_v)
        fire_gather(0, 0)

        def body(i, carry):
            for b in range(NBUF):
                j = i * NBUF + b
                jn = j + 1
                bn = (b + 1) % NBUF

                @pl.when(jnp.logical_and(j >= 1, jn < HCPT))
                def _():
                    wait_scatter(j - 1, bn)

                @pl.when(jn < HCPT)
                def _():
                    fire_gather(jn, bn)

                wait_gather(j, b)
                fire_scatter(j, b)
            return carry

        lax.fori_loop(0, HCPT // NBUF, body, 0)
        for b in range(NBUF):
            wait_scatter(HCPT - NBUF + b, (HCPT - NBUF + b) % NBUF)

    plsc.subcore_barrier()
    pltpu.sync_copy(acc.at[pl.ds(sid * RPT, RPT)],
                    out_hbm.at[cid, pl.ds(sid * RPT, RPT)])


def _tc1_body(degp_ref, x_ref, w1_ref, dinv_ref, hs_ref):
    deg = degp_ref[0, :N, 0:1] + degp_ref[1, :N, 0:1] + 1.0
    dinv = lax.rsqrt(deg)
    dinv_ref[...] = dinv
    h = jnp.dot(x_ref[...], w1_ref[...], preferred_element_type=jnp.float32)
    hs_ref[...] = h * dinv


def _tc2_body(p_ref, hs_ref, dinv_ref, b_ref, g_ref, be_ref, w2_ref, out_ref):
    dinv = dinv_ref[...]
    acc = p_ref[0, :N, :] + p_ref[1, :N, :] + hs_ref[...]
    y = acc * dinv + b_ref[...]
    mean = jnp.mean(y, axis=0, keepdims=True)
    var = jnp.mean((y - mean) * (y - mean), axis=0, keepdims=True)
    yn = (y - mean) * lax.rsqrt(var + 1e-5) * g_ref[...] + be_ref[...]
    yr = jnp.maximum(yn, 0.0)
    h2 = jnp.dot(yr, w2_ref[...], preferred_element_type=jnp.float32)
    out_ref[...] = h2 * dinv


def _tc3_body(p_ref, hs_ref, dinv_ref, b_ref, g_ref, be_ref, out_ref):
    acc = p_ref[0, :N, :] + p_ref[1, :N, :] + hs_ref[...]
    y = acc * dinv_ref[...] + b_ref[...]
    mean = jnp.mean(y, axis=0, keepdims=True)
    var = jnp.mean((y - mean) * (y - mean), axis=0, keepdims=True)
    out_ref[...] = (y - mean) * lax.rsqrt(var + 1e-5) * g_ref[...] + be_ref[...]


_f32 = jnp.float32
_tc1 = pl.pallas_call(
    _tc1_body,
    out_shape=[jax.ShapeDtypeStruct((N, 1), _f32),
               jax.ShapeDtypeStruct((N, D), _f32)],
)
_tc2 = pl.pallas_call(
    _tc2_body,
    out_shape=jax.ShapeDtypeStruct((N, D), _f32),
)
_tc3 = pl.pallas_call(
    _tc3_body,
    out_shape=jax.ShapeDtypeStruct((N, D), _f32),
)


def kernel(e_prev, edge_index, W1, b1, gamma1, beta1, W2, b2, gamma2, beta2):
    src = edge_index[0]
    dst = edge_index[1]
    pad = EPAD - E
    srcp = jnp.concatenate([src, jnp.zeros((pad,), jnp.int32)])
    srcp = srcp.reshape(NW, CPT, CHUNK)
    dstp = jnp.concatenate([dst, jnp.full((pad,), DUMMY, jnp.int32)])
    dstp = dstp.reshape(NW, CPT, CHUNK)

    ones_c = jnp.ones((CHUNK, DEGW), _f32)
    zeros_r = jnp.zeros((RPT, DEGW), _f32)
    zrows = jnp.zeros((RPT, D), _f32)

    degp = _deg_kernel(dstp, ones_c, zeros_r)
    dinv, hs1 = _tc1(degp, e_prev, W1)
    p1 = _agg_kernel(hs1, srcp, dstp, zrows)
    hs2 = _tc2(p1, hs1, dinv, b1.reshape(1, D), gamma1.reshape(1, D),
               beta1.reshape(1, D), W2)
    p2 = _agg_kernel(hs2, srcp, dstp, zrows)
    out = _tc3(p2, hs2, dinv, b2.reshape(1, D), gamma2.reshape(1, D),
               beta2.reshape(1, D))
    return out


# speedup vs baseline: 26.8809x; 2.9800x over previous
import functools

import jax
import jax.numpy as jnp
from jax import lax
from jax.experimental import pallas as pl
from jax.experimental.pallas import tpu as pltpu
from jax.experimental.pallas import tpu_sc as plsc

N = 10000
E = 320000
D = 128
NC = 2
NS = 16
NW = NC * NS
CHUNK = 128
CPT = 80
EPAD = NW * CPT * CHUNK
RPT = 640
NPAD = NS * RPT
DUMMY = N
DEGW = 128

_mesh = plsc.VectorSubcoreMesh(core_axis_name="c", subcore_axis_name="s")


@functools.partial(
    pl.kernel,
    out_type=jax.ShapeDtypeStruct((NC, NPAD, DEGW), jnp.float32),
    mesh=_mesh,
    scratch_types=[
        pltpu.VMEM((CPT, CHUNK), jnp.int32),
        pltpu.VMEM((CHUNK, DEGW), jnp.float32),
        pltpu.VMEM_SHARED((NPAD, DEGW), jnp.float32),
        pltpu.SemaphoreType.DMA,
    ],
)
def _deg_kernel(dst_hbm, ones_hbm, zeros_hbm, out_hbm, dst_v, ones_v, acc, sem):
    cid = lax.axis_index("c")
    sid = lax.axis_index("s")
    wid = cid * NS + sid
    pltpu.sync_copy(zeros_hbm, acc.at[pl.ds(sid * RPT, RPT)])
    pltpu.sync_copy(ones_hbm, ones_v)
    pltpu.sync_copy(dst_hbm.at[wid], dst_v)
    plsc.subcore_barrier()

    def fire(j, carry):
        pltpu.async_copy(ones_v, acc.at[dst_v.at[j]], sem, add=True)
        return carry

    lax.fori_loop(0, CPT, fire, 0)

    def drain(j, carry):
        pltpu.make_async_copy(ones_v, acc.at[dst_v.at[j]], sem).wait()
        return carry

    lax.fori_loop(0, CPT, drain, 0)
    plsc.subcore_barrier()
    pltpu.sync_copy(acc.at[pl.ds(sid * RPT, RPT)],
                    out_hbm.at[cid, pl.ds(sid * RPT, RPT)])


NBUF = 2
NHALF = 2
HCPT = CPT // NHALF


@functools.partial(
    pl.kernel,
    out_type=jax.ShapeDtypeStruct((NC, NPAD, D), jnp.float32),
    mesh=_mesh,
    scratch_types=(
        [pltpu.VMEM_SHARED((NPAD, D), jnp.float32)]
        + [pltpu.VMEM((HCPT, CHUNK), jnp.int32)] * 2
        + [pltpu.VMEM((CHUNK, D), jnp.float32)] * NBUF
        + [pltpu.SemaphoreType.DMA] * (2 * NBUF)
    ),
)
def _agg_kernel(hs_hbm, src_hbm, dst_hbm, zrows_hbm, out_hbm,
                acc, src_v, dst_v, r0, r1, g0, g1, s0, s1):
    rows = (r0, r1)
    gsem = (g0, g1)
    ssem = (s0, s1)
    cid = lax.axis_index("c")
    sid = lax.axis_index("s")
    wid = cid * NS + sid

    def fire_gather(j, b):
        pltpu.async_copy(hs_hbm.at[src_v.at[j]], rows[b], gsem[b])

    def wait_gather(j, b):
        pltpu.make_async_copy(hs_hbm.at[src_v.at[j]], rows[b], gsem[b]).wait()

    def fire_scatter(j, b):
        pltpu.async_copy(rows[b], acc.at[dst_v.at[j]], ssem[b], add=True)

    def wait_scatter(j, b):
        pltpu.make_async_copy(rows[b], acc.at[dst_v.at[j]], ssem[b]).wait()

    pltpu.sync_copy(zrows_hbm, acc.at[pl.ds(sid * RPT, RPT)])
    plsc.subcore_barrier()

    for h in range(NHALF):
        pltpu.sync_copy(src_hbm.at[wid, pl.ds(h * HCPT, HCPT)], src_v)
        pltpu.sync_copy(dst_hbm.at[wid, pl.ds(h * HCPT, HCPT)], dst_v)
        fire_gather(0, 0)

        def body(i, carry):
            for b in range(NBUF):
                j = i * NBUF + b
                jn = j + 1
                bn = (b + 1) % NBUF

                @pl.when(jnp.logical_and(j >= 1, jn < HCPT))
                def _():
                    wait_scatter(j - 1, bn)

                @pl.when(jn < HCPT)
                def _():
                    fire_gather(jn, bn)

                wait_gather(j, b)
                fire_scatter(j, b)
            return carry

        lax.fori_loop(0, HCPT // NBUF, body, 0)
        for b in range(NBUF):
            wait_scatter(HCPT - NBUF + b, (HCPT - NBUF + b) % NBUF)

    plsc.subcore_barrier()
    pltpu.sync_copy(acc.at[pl.ds(sid * RPT, RPT)],
                    out_hbm.at[cid, pl.ds(sid * RPT, RPT)])


def _tc1_body(degp_ref, x_ref, w1_ref, dinv_ref, hs_ref):
    deg = degp_ref[0, :N, 0:1] + degp_ref[1, :N, 0:1] + 1.0
    dinv = lax.rsqrt(deg)
    dinv_ref[...] = dinv
    h = jnp.dot(x_ref[...], w1_ref[...], preferred_element_type=jnp.float32)
    hs_ref[...] = h * dinv


def _tc2_body(p_ref, hs_ref, dinv_ref, b_ref, g_ref, be_ref, w2_ref, out_ref):
    dinv = dinv_ref[...]
    acc = p_ref[0, :N, :] + p_ref[1, :N, :] + hs_ref[...]
    y = acc * dinv + b_ref[...]
    mean = jnp.mean(y, axis=0, keepdims=True)
    var = jnp.mean((y - mean) * (y - mean), axis=0, keepdims=True)
    yn = (y - mean) * lax.rsqrt(var + 1e-5) * g_ref[...] + be_ref[...]
    yr = jnp.maximum(yn, 0.0)
    h2 = jnp.dot(yr, w2_ref[...], preferred_element_type=jnp.float32)
    out_ref[...] = h2 * dinv


def _tc3_body(p_ref, hs_ref, dinv_ref, b_ref, g_ref, be_ref, out_ref):
    acc = p_ref[0, :N, :] + p_ref[1, :N, :] + hs_ref[...]
    y = acc * dinv_ref[...] + b_ref[...]
    mean = jnp.mean(y, axis=0, keepdims=True)
    var = jnp.mean((y - mean) * (y - mean), axis=0, keepdims=True)
    out_ref[...] = (y - mean) * lax.rsqrt(var + 1e-5) * g_ref[...] + be_ref[...]


_f32 = jnp.float32
_tc1 = pl.pallas_call(
    _tc1_body,
    out_shape=[jax.ShapeDtypeStruct((N, 1), _f32),
               jax.ShapeDtypeStruct((N, D), _f32)],
)
_tc2 = pl.pallas_call(
    _tc2_body,
    out_shape=jax.ShapeDtypeStruct((N, D), _f32),
)
_tc3 = pl.pallas_call(
    _tc3_body,
    out_shape=jax.ShapeDtypeStruct((N, D), _f32),
)


def kernel(e_prev, edge_index, W1, b1, gamma1, beta1, W2, b2, gamma2, beta2):
    src = edge_index[0]
    dst = edge_index[1]
    pad = EPAD - E
    pad_src = (jnp.arange(pad, dtype=jnp.int32) * 37) % N
    pad_dst = N + (jnp.arange(pad, dtype=jnp.int32) % (NPAD - N))
    srcp = jnp.concatenate([src, pad_src]).reshape(NW, CPT, CHUNK)
    dstp = jnp.concatenate([dst, pad_dst]).reshape(NW, CPT, CHUNK)

    ones_c = jnp.ones((CHUNK, DEGW), _f32)
    zeros_r = jnp.zeros((RPT, DEGW), _f32)
    zrows = jnp.zeros((RPT, D), _f32)

    degp = _deg_kernel(dstp, ones_c, zeros_r)
    dinv, hs1 = _tc1(degp, e_prev, W1)
    p1 = _agg_kernel(hs1, srcp, dstp, zrows)
    hs2 = _tc2(p1, hs1, dinv, b1.reshape(1, D), gamma1.reshape(1, D),
               beta1.reshape(1, D), W2)
    p2 = _agg_kernel(hs2, srcp, dstp, zrows)
    out = _tc3(p2, hs2, dinv, b2.reshape(1, D), gamma2.reshape(1, D),
               beta2.reshape(1, D))
    return out


# speedup vs baseline: 27.0115x; 1.0049x over previous
import functools

import jax
import jax.numpy as jnp
from jax import lax
from jax.experimental import pallas as pl
from jax.experimental.pallas import tpu as pltpu
from jax.experimental.pallas import tpu_sc as plsc

N = 10000
E = 320000
D = 128
NC = 2
NS = 16
NW = NC * NS
CHUNK = 128
CPT = 80
EPAD = NW * CPT * CHUNK
RPT = 640
NPAD = NS * RPT
DUMMY = N
DEGW = 128

_mesh = plsc.VectorSubcoreMesh(core_axis_name="c", subcore_axis_name="s")


@functools.partial(
    pl.kernel,
    out_type=jax.ShapeDtypeStruct((NC, NPAD, DEGW), jnp.float32),
    mesh=_mesh,
    scratch_types=[
        pltpu.VMEM((CPT, CHUNK), jnp.int32),
        pltpu.VMEM((CHUNK, DEGW), jnp.float32),
        pltpu.VMEM_SHARED((NPAD, DEGW), jnp.float32),
        pltpu.SemaphoreType.DMA,
    ],
)
def _deg_kernel(dst_hbm, ones_hbm, zeros_hbm, out_hbm, dst_v, ones_v, acc, sem):
    cid = lax.axis_index("c")
    sid = lax.axis_index("s")
    wid = cid * NS + sid
    pltpu.sync_copy(zeros_hbm, acc.at[pl.ds(sid * RPT, RPT)])
    pltpu.sync_copy(ones_hbm, ones_v)
    pltpu.sync_copy(dst_hbm.at[wid], dst_v)
    plsc.subcore_barrier()

    def fire(j, carry):
        pltpu.async_copy(ones_v, acc.at[dst_v.at[j]], sem, add=True)
        return carry

    lax.fori_loop(0, CPT, fire, 0)

    def drain(j, carry):
        pltpu.make_async_copy(ones_v, acc.at[dst_v.at[j]], sem).wait()
        return carry

    lax.fori_loop(0, CPT, drain, 0)
    plsc.subcore_barrier()
    pltpu.sync_copy(acc.at[pl.ds(sid * RPT, RPT)],
                    out_hbm.at[cid, pl.ds(sid * RPT, RPT)])


NBUF = 2
NHALF = 2
HCPT = CPT // NHALF


@functools.partial(
    pl.kernel,
    out_type=jax.ShapeDtypeStruct((NC, NPAD, D), jnp.float32),
    mesh=_mesh,
    scratch_types=(
        [pltpu.VMEM_SHARED((NPAD, D), jnp.float32)]
        + [pltpu.VMEM((HCPT, CHUNK), jnp.int32)] * 2
        + [pltpu.VMEM((CHUNK, D), jnp.float32)] * NBUF
        + [pltpu.SemaphoreType.DMA] * (2 * NBUF)
    ),
)
def _agg_kernel(hs_hbm, src_hbm, dst_hbm, zrows_hbm, out_hbm,
                acc, src_v, dst_v, r0, r1, g0, g1, s0, s1):
    rows = (r0, r1)
    gsem = (g0, g1)
    ssem = (s0, s1)
    cid = lax.axis_index("c")
    sid = lax.axis_index("s")
    wid = cid * NS + sid

    def fire_gather(j, b):
        pltpu.async_copy(hs_hbm.at[src_v.at[j]], rows[b], gsem[b])

    def wait_gather(j, b):
        pltpu.make_async_copy(hs_hbm.at[src_v.at[j]], rows[b], gsem[b]).wait()

    def fire_scatter(j, b):
        pltpu.async_copy(rows[b], acc.at[dst_v.at[j]], ssem[b], add=True)

    def wait_scatter(j, b):
        pltpu.make_async_copy(rows[b], acc.at[dst_v.at[j]], ssem[b]).wait()

    pltpu.sync_copy(zrows_hbm, acc.at[pl.ds(sid * RPT, RPT)])
    plsc.subcore_barrier()

    for h in range(NHALF):
        pltpu.sync_copy(src_hbm.at[wid, pl.ds(h * HCPT, HCPT)], src_v)
        pltpu.sync_copy(dst_hbm.at[wid, pl.ds(h * HCPT, HCPT)], dst_v)
        fire_gather(0, 0)

        def body(i, carry):
            for b in range(NBUF):
                j = i * NBUF + b
                jn = j + 1
                bn = (b + 1) % NBUF

                @pl.when(jnp.logical_and(j >= 1, jn < HCPT))
                def _():
                    wait_scatter(j - 1, bn)

                @pl.when(jn < HCPT)
                def _():
                    fire_gather(jn, bn)

                wait_gather(j, b)
                fire_scatter(j, b)
            return carry

        lax.fori_loop(0, HCPT // NBUF, body, 0)
        for b in range(NBUF):
            wait_scatter(HCPT - NBUF + b, (HCPT - NBUF + b) % NBUF)

    plsc.subcore_barrier()
    pltpu.sync_copy(acc.at[pl.ds(sid * RPT, RPT)],
                    out_hbm.at[cid, pl.ds(sid * RPT, RPT)])


def _tc0_body(x_ref, w1_ref, h_ref):
    h_ref[...] = jnp.dot(x_ref[...], w1_ref[...],
                         preferred_element_type=jnp.float32)


def _tc1_body(degp_ref, h_ref, dinv_ref, hs_ref):
    deg = degp_ref[0, :N, 0:1] + degp_ref[1, :N, 0:1] + 1.0
    dinv = lax.rsqrt(deg)
    dinv_ref[...] = dinv
    hs_ref[...] = h_ref[...] * dinv


def _tc2_body(p_ref, hs_ref, dinv_ref, b_ref, g_ref, be_ref, w2_ref, out_ref):
    dinv = dinv_ref[...]
    acc = p_ref[0, :N, :] + p_ref[1, :N, :] + hs_ref[...]
    y = acc * dinv + b_ref[...]
    mean = jnp.mean(y, axis=0, keepdims=True)
    var = jnp.mean((y - mean) * (y - mean), axis=0, keepdims=True)
    yn = (y - mean) * lax.rsqrt(var + 1e-5) * g_ref[...] + be_ref[...]
    yr = jnp.maximum(yn, 0.0)
    h2 = jnp.dot(yr, w2_ref[...], preferred_element_type=jnp.float32)
    out_ref[...] = h2 * dinv


def _tc3_body(p_ref, hs_ref, dinv_ref, b_ref, g_ref, be_ref, out_ref):
    acc = p_ref[0, :N, :] + p_ref[1, :N, :] + hs_ref[...]
    y = acc * dinv_ref[...] + b_ref[...]
    mean = jnp.mean(y, axis=0, keepdims=True)
    var = jnp.mean((y - mean) * (y - mean), axis=0, keepdims=True)
    out_ref[...] = (y - mean) * lax.rsqrt(var + 1e-5) * g_ref[...] + be_ref[...]


_f32 = jnp.float32
_tc0 = pl.pallas_call(
    _tc0_body,
    out_shape=jax.ShapeDtypeStruct((N, D), _f32),
)
_tc1 = pl.pallas_call(
    _tc1_body,
    out_shape=[jax.ShapeDtypeStruct((N, 1), _f32),
               jax.ShapeDtypeStruct((N, D), _f32)],
)
_tc2 = pl.pallas_call(
    _tc2_body,
    out_shape=jax.ShapeDtypeStruct((N, D), _f32),
)
_tc3 = pl.pallas_call(
    _tc3_body,
    out_shape=jax.ShapeDtypeStruct((N, D), _f32),
)


def kernel(e_prev, edge_index, W1, b1, gamma1, beta1, W2, b2, gamma2, beta2):
    src = edge_index[0]
    dst = edge_index[1]
    pad = EPAD - E
    pad_src = (jnp.arange(pad, dtype=jnp.int32) * 37) % N
    pad_dst = N + (jnp.arange(pad, dtype=jnp.int32) % (NPAD - N))
    srcp = jnp.concatenate([src, pad_src]).reshape(NW, CPT, CHUNK)
    dstp = jnp.concatenate([dst, pad_dst]).reshape(NW, CPT, CHUNK)

    ones_c = jnp.ones((CHUNK, DEGW), _f32)
    zeros_r = jnp.zeros((RPT, DEGW), _f32)
    zrows = jnp.zeros((RPT, D), _f32)

    h1 = _tc0(e_prev, W1)
    degp = _deg_kernel(dstp, ones_c, zeros_r)
    dinv, hs1 = _tc1(degp, h1)
    p1 = _agg_kernel(hs1, srcp, dstp, zrows)
    hs2 = _tc2(p1, hs1, dinv, b1.reshape(1, D), gamma1.reshape(1, D),
               beta1.reshape(1, D), W2)
    p2 = _agg_kernel(hs2, srcp, dstp, zrows)
    out = _tc3(p2, hs2, dinv, b2.reshape(1, D), gamma2.reshape(1, D),
               beta2.reshape(1, D))
    return out


# speedup vs baseline: 27.9963x; 1.0365x over previous
import functools

import jax
import jax.numpy as jnp
from jax import lax
from jax.experimental import pallas as pl
from jax.experimental.pallas import tpu as pltpu
from jax.experimental.pallas import tpu_sc as plsc

N = 10000
E = 320000
D = 128
NC = 2
NS = 16
NW = NC * NS
CHUNK = 128
CPT = 80
EPT = CPT * CHUNK
EPAD = NW * EPT
RPT = 640
NPAD = NS * RPT
DUMMY = N
DEGW = 128

_mesh = plsc.VectorSubcoreMesh(core_axis_name="c", subcore_axis_name="s")


@functools.partial(
    pl.kernel,
    out_type=jax.ShapeDtypeStruct((NC, NPAD, DEGW), jnp.float32),
    mesh=_mesh,
    scratch_types=[
        pltpu.VMEM((CPT, CHUNK), jnp.int32),
        pltpu.VMEM((CHUNK, DEGW), jnp.float32),
        pltpu.VMEM_SHARED((NPAD, DEGW), jnp.float32),
        pltpu.SemaphoreType.DMA,
    ],
)
def _deg_kernel(dst_hbm, ones_hbm, zeros_hbm, out_hbm, dst_v, ones_v, acc, sem):
    cid = lax.axis_index("c")
    sid = lax.axis_index("s")
    wid = cid * NS + sid
    pltpu.sync_copy(zeros_hbm, acc.at[pl.ds(sid * RPT, RPT)])
    pltpu.sync_copy(ones_hbm, ones_v)
    pltpu.sync_copy(dst_hbm.at[wid], dst_v)
    plsc.subcore_barrier()

    def fire(j, carry):
        pltpu.async_copy(ones_v, acc.at[dst_v.at[j]], sem, add=True)
        return carry

    lax.fori_loop(0, CPT, fire, 0)

    def drain(j, carry):
        pltpu.make_async_copy(ones_v, acc.at[dst_v.at[j]], sem).wait()
        return carry

    lax.fori_loop(0, CPT, drain, 0)
    plsc.subcore_barrier()
    pltpu.sync_copy(acc.at[pl.ds(sid * RPT, RPT)],
                    out_hbm.at[cid, pl.ds(sid * RPT, RPT)])


ACHUNK = 64
ACPT = EPT // ACHUNK
NBUF = 4
NHALF = 4
HCPT = ACPT // NHALF


@functools.partial(
    pl.kernel,
    out_type=jax.ShapeDtypeStruct((NC, NPAD, D), jnp.float32),
    mesh=_mesh,
    scratch_types=(
        [pltpu.VMEM_SHARED((NPAD, D), jnp.float32)]
        + [pltpu.VMEM((HCPT, ACHUNK), jnp.int32)] * 2
        + [pltpu.VMEM((ACHUNK, D), jnp.float32)] * NBUF
        + [pltpu.SemaphoreType.DMA] * (2 * NBUF)
    ),
)
def _agg_kernel(hs_hbm, src_hbm, dst_hbm, zrows_hbm, out_hbm,
                acc, src_v, dst_v, r0, r1, r2, r3,
                g0, g1, g2, g3, s0, s1, s2, s3):
    rows = (r0, r1, r2, r3)
    gsem = (g0, g1, g2, g3)
    ssem = (s0, s1, s2, s3)
    cid = lax.axis_index("c")
    sid = lax.axis_index("s")
    wid = cid * NS + sid

    def fire_gather(j, b):
        pltpu.async_copy(hs_hbm.at[src_v.at[j]], rows[b], gsem[b])

    def wait_gather(j, b):
        pltpu.make_async_copy(hs_hbm.at[src_v.at[j]], rows[b], gsem[b]).wait()

    def fire_scatter(j, b):
        pltpu.async_copy(rows[b], acc.at[dst_v.at[j]], ssem[b], add=True)

    def wait_scatter(j, b):
        pltpu.make_async_copy(rows[b], acc.at[dst_v.at[j]], ssem[b]).wait()

    pltpu.sync_copy(zrows_hbm, acc.at[pl.ds(sid * RPT, RPT)])
    plsc.subcore_barrier()

    for h in range(NHALF):
        pltpu.sync_copy(src_hbm.at[wid, pl.ds(h * HCPT, HCPT)], src_v)
        pltpu.sync_copy(dst_hbm.at[wid, pl.ds(h * HCPT, HCPT)], dst_v)
        for c in range(NBUF - 1):
            fire_gather(c, c)

        def body(i, carry):
            for b in range(NBUF):
                j = i * NBUF + b
                jn = j + NBUF - 1
                bn = (b + NBUF - 1) % NBUF

                @pl.when(jnp.logical_and(j >= 1, jn < HCPT))
                def _():
                    wait_scatter(j - 1, bn)

                @pl.when(jn < HCPT)
                def _():
                    fire_gather(jn, bn)

                wait_gather(j, b)
                fire_scatter(j, b)
            return carry

        lax.fori_loop(0, HCPT // NBUF, body, 0)
        for b in range(NBUF):
            wait_scatter(HCPT - NBUF + b, (HCPT - NBUF + b) % NBUF)

    plsc.subcore_barrier()
    pltpu.sync_copy(acc.at[pl.ds(sid * RPT, RPT)],
                    out_hbm.at[cid, pl.ds(sid * RPT, RPT)])


def _tc0_body(x_ref, w1_ref, h_ref):
    h_ref[...] = jnp.dot(x_ref[...], w1_ref[...],
                         preferred_element_type=jnp.float32)


def _tc1_body(degp_ref, h_ref, dinv_ref, hs_ref):
    deg = degp_ref[0, :N, 0:1] + degp_ref[1, :N, 0:1] + 1.0
    dinv = lax.rsqrt(deg)
    dinv_ref[...] = dinv
    hs_ref[...] = h_ref[...] * dinv


def _tc2_body(p_ref, hs_ref, dinv_ref, b_ref, g_ref, be_ref, w2_ref, out_ref):
    dinv = dinv_ref[...]
    acc = p_ref[0, :N, :] + p_ref[1, :N, :] + hs_ref[...]
    y = acc * dinv + b_ref[...]
    mean = jnp.mean(y, axis=0, keepdims=True)
    var = jnp.mean((y - mean) * (y - mean), axis=0, keepdims=True)
    yn = (y - mean) * lax.rsqrt(var + 1e-5) * g_ref[...] + be_ref[...]
    yr = jnp.maximum(yn, 0.0)
    h2 = jnp.dot(yr, w2_ref[...], preferred_element_type=jnp.float32)
    out_ref[...] = h2 * dinv


def _tc3_body(p_ref, hs_ref, dinv_ref, b_ref, g_ref, be_ref, out_ref):
    acc = p_ref[0, :N, :] + p_ref[1, :N, :] + hs_ref[...]
    y = acc * dinv_ref[...] + b_ref[...]
    mean = jnp.mean(y, axis=0, keepdims=True)
    var = jnp.mean((y - mean) * (y - mean), axis=0, keepdims=True)
    out_ref[...] = (y - mean) * lax.rsqrt(var + 1e-5) * g_ref[...] + be_ref[...]


_f32 = jnp.float32
_tc0 = pl.pallas_call(
    _tc0_body,
    out_shape=jax.ShapeDtypeStruct((N, D), _f32),
)
_tc1 = pl.pallas_call(
    _tc1_body,
    out_shape=[jax.ShapeDtypeStruct((N, 1), _f32),
               jax.ShapeDtypeStruct((N, D), _f32)],
)
_tc2 = pl.pallas_call(
    _tc2_body,
    out_shape=jax.ShapeDtypeStruct((N, D), _f32),
)
_tc3 = pl.pallas_call(
    _tc3_body,
    out_shape=jax.ShapeDtypeStruct((N, D), _f32),
)


def kernel(e_prev, edge_index, W1, b1, gamma1, beta1, W2, b2, gamma2, beta2):
    src = edge_index[0]
    dst = edge_index[1]
    pad = EPAD - E
    pad_src = (jnp.arange(pad, dtype=jnp.int32) * 37) % N
    pad_dst = N + (jnp.arange(pad, dtype=jnp.int32) % (NPAD - N))
    srcp = jnp.concatenate([src, pad_src]).reshape(NW, CPT, CHUNK)
    dstp = jnp.concatenate([dst, pad_dst]).reshape(NW, CPT, CHUNK)

    ones_c = jnp.ones((CHUNK, DEGW), _f32)
    zeros_r = jnp.zeros((RPT, DEGW), _f32)
    zrows = jnp.zeros((RPT, D), _f32)

    srcp_a = srcp.reshape(NW, ACPT, ACHUNK)
    dstp_a = dstp.reshape(NW, ACPT, ACHUNK)

    h1 = _tc0(e_prev, W1)
    degp = _deg_kernel(dstp, ones_c, zeros_r)
    dinv, hs1 = _tc1(degp, h1)
    p1 = _agg_kernel(hs1, srcp_a, dstp_a, zrows)
    hs2 = _tc2(p1, hs1, dinv, b1.reshape(1, D), gamma1.reshape(1, D),
               beta1.reshape(1, D), W2)
    p2 = _agg_kernel(hs2, srcp_a, dstp_a, zrows)
    out = _tc3(p2, hs2, dinv, b2.reshape(1, D), gamma2.reshape(1, D),
               beta2.reshape(1, D))
    return out
